# baseline (device time: 187408 ns/iter reference)
import jax
import jax.numpy as jnp
from jax import lax
from jax.experimental import pallas as pl
from jax.experimental.pallas import tpu as pltpu

N_DEV = 4
SQ = 1024
SKV = 1024
H_LOCAL = 8
DH = 128
D_MODEL = 1024
WINDOW = 128
SCALE = 0.08838834764831843


def kernel(x, Wq, K_ext, V_ext, Wo):
    my_pos = lax.axis_index("i")

    x2 = x[0]
    Wq_loc = lax.dynamic_slice(Wq, (0, my_pos * H_LOCAL * DH), (D_MODEL, H_LOCAL * DH))
    Wo_loc = lax.dynamic_slice(Wo, (my_pos * H_LOCAL * DH, 0), (H_LOCAL * DH, D_MODEL))
    K2 = jnp.transpose(K_ext[0], (1, 0, 2))
    V2 = jnp.transpose(V_ext[0], (1, 0, 2))

    def body(x_ref, wq_ref, k_ref, v_ref, wo_ref, out_ref,
             comm_ref, send_sems, recv_sems):
        me = lax.axis_index("i")
        left = (me - 1) % N_DEV
        right = (me + 1) % N_DEV

        barrier_sem = pltpu.get_barrier_semaphore()
        for nbr in (left, right):
            pl.semaphore_signal(
                barrier_sem, inc=1,
                device_id=(nbr,), device_id_type=pl.DeviceIdType.MESH,
            )
        pl.semaphore_wait(barrier_sem, 2)

        xv = x_ref[:, :]
        qi = lax.broadcasted_iota(jnp.int32, (SQ, SKV), 0)
        ki = lax.broadcasted_iota(jnp.int32, (SQ, SKV), 1)
        mask = jnp.abs(qi - ki) <= WINDOW

        acc = jnp.zeros((SQ, D_MODEL), jnp.float32)
        for h in range(H_LOCAL):
            q = jnp.dot(xv, wq_ref[:, h * DH:(h + 1) * DH],
                        preferred_element_type=jnp.float32)
            s = lax.dot_general(q, k_ref[h],
                                (((1,), (1,)), ((), ())),
                                preferred_element_type=jnp.float32) * SCALE
            s = jnp.where(mask, s, -1e9)
            m = jnp.max(s, axis=1, keepdims=True)
            w = jnp.exp(s - m)
            w = w / jnp.sum(w, axis=1, keepdims=True)
            ctx = jnp.dot(w, v_ref[h], preferred_element_type=jnp.float32)
            acc = acc + jnp.dot(ctx, wo_ref[h * DH:(h + 1) * DH, :],
                                preferred_element_type=jnp.float32)

        comm_ref[0, :, :] = acc

        for h in range(N_DEV - 1):
            rdma = pltpu.make_async_remote_copy(
                src_ref=comm_ref.at[h],
                dst_ref=comm_ref.at[h + 1],
                send_sem=send_sems.at[h],
                recv_sem=recv_sems.at[h],
                device_id=(right,),
                device_id_type=pl.DeviceIdType.MESH,
            )
            rdma.start()
            rdma.wait()
            acc = acc + comm_ref[h + 1, :, :]

        out_ref[:, :] = acc

    out = pl.pallas_call(
        body,
        out_shape=jax.ShapeDtypeStruct((SQ, D_MODEL), jnp.float32),
        in_specs=[pl.BlockSpec(memory_space=pltpu.VMEM)] * 5,
        out_specs=pl.BlockSpec(memory_space=pltpu.VMEM),
        scratch_shapes=[
            pltpu.VMEM((N_DEV, SQ, D_MODEL), jnp.float32),
            pltpu.SemaphoreType.DMA((N_DEV - 1,)),
            pltpu.SemaphoreType.DMA((N_DEV - 1,)),
        ],
        compiler_params=pltpu.CompilerParams(collective_id=0),
    )(x2, Wq_loc, K2, V2, Wo_loc)
    return out[None]


# device time: 83118 ns/iter; 2.2547x vs baseline; 2.2547x over previous
import jax
import jax.numpy as jnp
from jax import lax
from jax.experimental import pallas as pl
from jax.experimental.pallas import tpu as pltpu

N_DEV = 4
SQ = 1024
SKV = 1024
H_LOCAL = 8
DH = 128
D_MODEL = 1024
WINDOW = 128
SCALE = 0.08838834764831843
CHUNK = SQ // N_DEV


def kernel(x, Wq, K_ext, V_ext, Wo):
    my_pos = lax.axis_index("i")

    x2 = x[0]
    Wq_loc = lax.dynamic_slice(Wq, (0, my_pos * H_LOCAL * DH), (D_MODEL, H_LOCAL * DH))
    Wo_loc = lax.dynamic_slice(Wo, (my_pos * H_LOCAL * DH, 0), (H_LOCAL * DH, D_MODEL))
    K2 = jnp.transpose(K_ext[0], (1, 0, 2))
    V2 = jnp.transpose(V_ext[0], (1, 0, 2))

    def body(x_ref, wq_ref, k_ref, v_ref, wo_ref, out_ref,
             partial_ref, rs_recv_ref, ag_ref,
             rs_send_sems, rs_recv_sems, ag_send_sems, ag_recv_sems):
        me = lax.axis_index("i")

        barrier_sem = pltpu.get_barrier_semaphore()
        for p in range(N_DEV):
            @pl.when(p != me)
            def _():
                pl.semaphore_signal(
                    barrier_sem, inc=1,
                    device_id=(p,), device_id_type=pl.DeviceIdType.MESH,
                )
        pl.semaphore_wait(barrier_sem, N_DEV - 1)

        for c in range(N_DEV):
            lo = max(0, c * CHUNK - WINDOW)
            hi = min(SKV, (c + 1) * CHUNK + WINDOW)
            w_kv = hi - lo

            xc = x_ref[c * CHUNK:(c + 1) * CHUNK, :]
            qc = jnp.dot(xc, wq_ref[:, :], preferred_element_type=jnp.float32)

            qi = lax.broadcasted_iota(jnp.int32, (CHUNK, w_kv), 0) + c * CHUNK
            ki = lax.broadcasted_iota(jnp.int32, (CHUNK, w_kv), 1) + lo
            mask = jnp.abs(qi - ki) <= WINDOW

            ctxs = []
            for h in range(H_LOCAL):
                q = qc[:, h * DH:(h + 1) * DH]
                s = lax.dot_general(q, k_ref[h, lo:hi, :],
                                    (((1,), (1,)), ((), ())),
                                    preferred_element_type=jnp.float32) * SCALE
                s = jnp.where(mask, s, -1e9)
                m = jnp.max(s, axis=1, keepdims=True)
                w = jnp.exp(s - m)
                w = w / jnp.sum(w, axis=1, keepdims=True)
                ctxs.append(jnp.dot(w, v_ref[h, lo:hi, :],
                                    preferred_element_type=jnp.float32))
            ctx = jnp.concatenate(ctxs, axis=1)
            acc_c = jnp.dot(ctx, wo_ref[:, :], preferred_element_type=jnp.float32)

            @pl.when(c == me)
            def _():
                rs_recv_ref[c, :, :] = acc_c

            @pl.when(c != me)
            def _():
                partial_ref[c, :, :] = acc_c
                rdma = pltpu.make_async_remote_copy(
                    src_ref=partial_ref.at[c],
                    dst_ref=rs_recv_ref.at[me],
                    send_sem=rs_send_sems.at[c],
                    recv_sem=rs_recv_sems.at[me],
                    device_id=(c,),
                    device_id_type=pl.DeviceIdType.MESH,
                )
                rdma.start()

        for j in range(N_DEV):
            @pl.when(j != me)
            def _():
                pltpu.make_async_remote_copy(
                    src_ref=rs_recv_ref.at[j],
                    dst_ref=rs_recv_ref.at[j],
                    send_sem=rs_send_sems.at[j],
                    recv_sem=rs_recv_sems.at[j],
                    device_id=(j,),
                    device_id_type=pl.DeviceIdType.MESH,
                ).wait_recv()

        red = rs_recv_ref[0, :, :]
        for j in range(1, N_DEV):
            red = red + rs_recv_ref[j, :, :]

        for j in range(N_DEV):
            @pl.when(j == me)
            def _():
                ag_ref[j, :, :] = red

        for j in range(N_DEV):
            @pl.when(j != me)
            def _():
                pltpu.make_async_remote_copy(
                    src_ref=ag_ref.at[me],
                    dst_ref=ag_ref.at[me],
                    send_sem=ag_send_sems.at[j],
                    recv_sem=ag_recv_sems.at[me],
                    device_id=(j,),
                    device_id_type=pl.DeviceIdType.MESH,
                ).start()

        for j in range(N_DEV):
            @pl.when(j != me)
            def _():
                pltpu.make_async_remote_copy(
                    src_ref=ag_ref.at[j],
                    dst_ref=ag_ref.at[j],
                    send_sem=ag_send_sems.at[j],
                    recv_sem=ag_recv_sems.at[j],
                    device_id=(j,),
                    device_id_type=pl.DeviceIdType.MESH,
                ).wait_recv()

        out_ref[:, :] = ag_ref[:, :, :].reshape(SQ, D_MODEL)

        for j in range(N_DEV):
            @pl.when(j != me)
            def _():
                pltpu.make_async_remote_copy(
                    src_ref=partial_ref.at[j],
                    dst_ref=partial_ref.at[j],
                    send_sem=rs_send_sems.at[j],
                    recv_sem=rs_recv_sems.at[j],
                    device_id=(j,),
                    device_id_type=pl.DeviceIdType.MESH,
                ).wait_send()
                pltpu.make_async_remote_copy(
                    src_ref=ag_ref.at[j],
                    dst_ref=ag_ref.at[j],
                    send_sem=ag_send_sems.at[j],
                    recv_sem=ag_recv_sems.at[j],
                    device_id=(j,),
                    device_id_type=pl.DeviceIdType.MESH,
                ).wait_send()

    out = pl.pallas_call(
        body,
        out_shape=jax.ShapeDtypeStruct((SQ, D_MODEL), jnp.float32),
        in_specs=[pl.BlockSpec(memory_space=pltpu.VMEM)] * 5,
        out_specs=pl.BlockSpec(memory_space=pltpu.VMEM),
        scratch_shapes=[
            pltpu.VMEM((N_DEV, CHUNK, D_MODEL), jnp.float32),
            pltpu.VMEM((N_DEV, CHUNK, D_MODEL), jnp.float32),
            pltpu.VMEM((N_DEV, CHUNK, D_MODEL), jnp.float32),
            pltpu.SemaphoreType.DMA((N_DEV,)),
            pltpu.SemaphoreType.DMA((N_DEV,)),
            pltpu.SemaphoreType.DMA((N_DEV,)),
            pltpu.SemaphoreType.DMA((N_DEV,)),
        ],
        compiler_params=pltpu.CompilerParams(collective_id=0),
    )(x2, Wq_loc, K2, V2, Wo_loc)
    return out[None]


# device time: 76818 ns/iter; 2.4396x vs baseline; 1.0820x over previous
import jax
import jax.numpy as jnp
from jax import lax
from jax.experimental import pallas as pl
from jax.experimental.pallas import tpu as pltpu

N_DEV = 4
SQ = 1024
SKV = 1024
H_LOCAL = 8
DH = 128
D_MODEL = 1024
WINDOW = 128
SCALE = 0.08838834764831843
CHUNK = SQ // N_DEV
KV_W = CHUNK + 2 * WINDOW


def kernel(x, Wq, K_ext, V_ext, Wo):
    my_pos = lax.axis_index("i")

    x2 = x[0]
    Wq_loc = lax.dynamic_slice(Wq, (0, my_pos * H_LOCAL * DH), (D_MODEL, H_LOCAL * DH))
    Wo_loc = lax.dynamic_slice(Wo, (my_pos * H_LOCAL * DH, 0), (H_LOCAL * DH, D_MODEL))
    K2 = K_ext[0]
    V2 = V_ext[0]

    def body(x_ref, wq_ref, k_ref, v_ref, wo_ref, out_ref,
             partial_ref, rs_recv_ref,
             rs_send_sems, rs_recv_sems, ag_send_sems, ag_recv_sems):
        me = lax.axis_index("i")

        barrier_sem = pltpu.get_barrier_semaphore()
        for p in range(N_DEV):
            @pl.when(p != me)
            def _():
                pl.semaphore_signal(
                    barrier_sem, inc=1,
                    device_id=(p,), device_id_type=pl.DeviceIdType.MESH,
                )
        pl.semaphore_wait(barrier_sem, N_DEV - 1)

        for step in range(N_DEV):
            c = (me + 1 + step) % N_DEV
            row0 = c * CHUNK
            lo = jnp.clip(row0 - WINDOW, 0, SKV - KV_W)

            xc = x_ref[pl.ds(row0, CHUNK), :]
            qc = jnp.dot(xc, wq_ref[:, :], preferred_element_type=jnp.float32)

            qi = lax.broadcasted_iota(jnp.int32, (CHUNK, KV_W), 0) + row0
            ki = lax.broadcasted_iota(jnp.int32, (CHUNK, KV_W), 1) + lo
            mask = jnp.abs(qi - ki) <= WINDOW

            ctxs = []
            for h in range(H_LOCAL):
                q = qc[:, h * DH:(h + 1) * DH]
                k = k_ref[pl.ds(lo, KV_W), h, :]
                s = lax.dot_general(q, k,
                                    (((1,), (1,)), ((), ())),
                                    preferred_element_type=jnp.float32) * SCALE
                s = jnp.where(mask, s, -1e9)
                m = jnp.max(s, axis=1, keepdims=True)
                w = jnp.exp(s - m)
                w = w / jnp.sum(w, axis=1, keepdims=True)
                ctxs.append(jnp.dot(w, v_ref[pl.ds(lo, KV_W), h, :],
                                    preferred_element_type=jnp.float32))
            ctx = jnp.concatenate(ctxs, axis=1)
            acc_c = jnp.dot(ctx, wo_ref[:, :], preferred_element_type=jnp.float32)

            if step < N_DEV - 1:
                partial_ref[step, :, :] = acc_c
                pltpu.make_async_remote_copy(
                    src_ref=partial_ref.at[step],
                    dst_ref=rs_recv_ref.at[me],
                    send_sem=rs_send_sems.at[step],
                    recv_sem=rs_recv_sems.at[me],
                    device_id=(c,),
                    device_id_type=pl.DeviceIdType.MESH,
                ).start()
            else:
                for j in range(N_DEV):
                    @pl.when(j == me)
                    def _():
                        rs_recv_ref[j, :, :] = acc_c

        for j in range(N_DEV):
            @pl.when(j != me)
            def _():
                pltpu.make_async_remote_copy(
                    src_ref=rs_recv_ref.at[j],
                    dst_ref=rs_recv_ref.at[j],
                    send_sem=rs_send_sems.at[0],
                    recv_sem=rs_recv_sems.at[j],
                    device_id=(j,),
                    device_id_type=pl.DeviceIdType.MESH,
                ).wait_recv()

        red = rs_recv_ref[0, :, :]
        for j in range(1, N_DEV):
            red = red + rs_recv_ref[j, :, :]

        for j in range(N_DEV):
            @pl.when(j == me)
            def _():
                out_ref[j * CHUNK:(j + 1) * CHUNK, :] = red

        for j in range(N_DEV):
            @pl.when(j != me)
            def _():
                pltpu.make_async_remote_copy(
                    src_ref=out_ref.at[pl.ds(me * CHUNK, CHUNK)],
                    dst_ref=out_ref.at[pl.ds(me * CHUNK, CHUNK)],
                    send_sem=ag_send_sems.at[j],
                    recv_sem=ag_recv_sems.at[me],
                    device_id=(j,),
                    device_id_type=pl.DeviceIdType.MESH,
                ).start()

        for j in range(N_DEV):
            @pl.when(j != me)
            def _():
                pltpu.make_async_remote_copy(
                    src_ref=out_ref.at[pl.ds(j * CHUNK, CHUNK)],
                    dst_ref=out_ref.at[pl.ds(j * CHUNK, CHUNK)],
                    send_sem=ag_send_sems.at[j],
                    recv_sem=ag_recv_sems.at[j],
                    device_id=(j,),
                    device_id_type=pl.DeviceIdType.MESH,
                ).wait_recv()

        for s in range(N_DEV - 1):
            pltpu.make_async_remote_copy(
                src_ref=partial_ref.at[s],
                dst_ref=partial_ref.at[s],
                send_sem=rs_send_sems.at[s],
                recv_sem=rs_recv_sems.at[0],
                device_id=(0,),
                device_id_type=pl.DeviceIdType.MESH,
            ).wait_send()
        for j in range(N_DEV):
            @pl.when(j != me)
            def _():
                pltpu.make_async_remote_copy(
                    src_ref=out_ref.at[pl.ds(0, CHUNK)],
                    dst_ref=out_ref.at[pl.ds(0, CHUNK)],
                    send_sem=ag_send_sems.at[j],
                    recv_sem=ag_recv_sems.at[j],
                    device_id=(j,),
                    device_id_type=pl.DeviceIdType.MESH,
                ).wait_send()

    out = pl.pallas_call(
        body,
        out_shape=jax.ShapeDtypeStruct((SQ, D_MODEL), jnp.float32),
        in_specs=[pl.BlockSpec(memory_space=pltpu.VMEM)] * 5,
        out_specs=pl.BlockSpec(memory_space=pltpu.VMEM),
        scratch_shapes=[
            pltpu.VMEM((N_DEV - 1, CHUNK, D_MODEL), jnp.float32),
            pltpu.VMEM((N_DEV, CHUNK, D_MODEL), jnp.float32),
            pltpu.SemaphoreType.DMA((N_DEV - 1,)),
            pltpu.SemaphoreType.DMA((N_DEV,)),
            pltpu.SemaphoreType.DMA((N_DEV,)),
            pltpu.SemaphoreType.DMA((N_DEV,)),
        ],
        compiler_params=pltpu.CompilerParams(collective_id=0),
    )(x2, Wq_loc, K2, V2, Wo_loc)
    return out[None]


# device time: 50199 ns/iter; 3.7333x vs baseline; 1.5303x over previous
import jax
import jax.numpy as jnp
from jax import lax
from jax.experimental import pallas as pl
from jax.experimental.pallas import tpu as pltpu

N_DEV = 4
SQ = 1024
SKV = 1024
H_LOCAL = 8
DH = 128
D_MODEL = 1024
H_DIM = H_LOCAL * DH
WINDOW = 128
SCALE = 0.08838834764831843
CHUNK = SQ // N_DEV
KV_W = CHUNK + 2 * WINDOW


def kernel(x, Wq, K_ext, V_ext, Wo):
    def body(x_ref, wq_hbm, k_hbm, v_hbm, wo_hbm, out_ref,
             wq_ref, wo_ref, k_ref, v_ref, partial_ref, rs_recv_ref, ag_ref,
             load_sems, rs_send_sems, rs_recv_sems, ag_send_sems, ag_recv_sems):
        me = lax.axis_index("i")

        barrier_sem = pltpu.get_barrier_semaphore()
        for p in range(N_DEV):
            @pl.when(p != me)
            def _():
                pl.semaphore_signal(
                    barrier_sem, inc=1,
                    device_id=(p,), device_id_type=pl.DeviceIdType.MESH,
                )

        loads = []
        loads.append(pltpu.make_async_copy(
            wq_hbm.at[:, pl.ds(me * H_DIM, H_DIM)], wq_ref, load_sems.at[0]))
        loads.append(pltpu.make_async_copy(
            wo_hbm.at[pl.ds(me * H_DIM, H_DIM), :], wo_ref, load_sems.at[1]))
        for h in range(H_LOCAL):
            loads.append(pltpu.make_async_copy(
                k_hbm.at[0, :, h, :], k_ref.at[h], load_sems.at[2 + h]))
            loads.append(pltpu.make_async_copy(
                v_hbm.at[0, :, h, :], v_ref.at[h], load_sems.at[10 + h]))
        for cp in loads:
            cp.start()
        for cp in loads:
            cp.wait()

        for step in range(N_DEV):
            c = (me + 1 + step) % N_DEV
            row0 = c * CHUNK
            lo = jnp.clip(row0 - WINDOW, 0, SKV - KV_W)

            xc = x_ref[pl.ds(row0, CHUNK), :]
            qc = jnp.dot(xc, wq_ref[:, :], preferred_element_type=jnp.float32)

            qi = lax.broadcasted_iota(jnp.int32, (CHUNK, KV_W), 0) + row0
            ki = lax.broadcasted_iota(jnp.int32, (CHUNK, KV_W), 1) + lo
            mask = jnp.abs(qi - ki) <= WINDOW

            ctxs = []
            for h in range(H_LOCAL):
                q = qc[:, h * DH:(h + 1) * DH]
                k = k_ref[h, pl.ds(lo, KV_W), :]
                s = lax.dot_general(q, k,
                                    (((1,), (1,)), ((), ())),
                                    preferred_element_type=jnp.float32) * SCALE
                s = jnp.where(mask, s, -1e9)
                m = jnp.max(s, axis=1, keepdims=True)
                w = jnp.exp(s - m)
                w = w / jnp.sum(w, axis=1, keepdims=True)
                ctxs.append(jnp.dot(w, v_ref[h, pl.ds(lo, KV_W), :],
                                    preferred_element_type=jnp.float32))
            ctx = jnp.concatenate(ctxs, axis=1)
            acc_c = jnp.dot(ctx, wo_ref[:, :], preferred_element_type=jnp.float32)

            if step == 0:
                pl.semaphore_wait(barrier_sem, N_DEV - 1)

            if step < N_DEV - 1:
                partial_ref[step, :, :] = acc_c.astype(jnp.bfloat16)
                pltpu.make_async_remote_copy(
                    src_ref=partial_ref.at[step],
                    dst_ref=rs_recv_ref.at[me],
                    send_sem=rs_send_sems.at[step],
                    recv_sem=rs_recv_sems.at[me],
                    device_id=(c,),
                    device_id_type=pl.DeviceIdType.MESH,
                ).start()
            else:
                for j in range(N_DEV):
                    @pl.when(j == me)
                    def _():
                        rs_recv_ref[j, :, :] = acc_c.astype(jnp.bfloat16)

        for j in range(N_DEV):
            @pl.when(j != me)
            def _():
                pltpu.make_async_remote_copy(
                    src_ref=rs_recv_ref.at[j],
                    dst_ref=rs_recv_ref.at[j],
                    send_sem=rs_send_sems.at[0],
                    recv_sem=rs_recv_sems.at[j],
                    device_id=(j,),
                    device_id_type=pl.DeviceIdType.MESH,
                ).wait_recv()

        red = rs_recv_ref[0, :, :].astype(jnp.float32)
        for j in range(1, N_DEV):
            red = red + rs_recv_ref[j, :, :].astype(jnp.float32)

        for j in range(N_DEV):
            @pl.when(j == me)
            def _():
                ag_ref[j, :, :] = red.astype(jnp.bfloat16)

        for j in range(N_DEV):
            @pl.when(j != me)
            def _():
                pltpu.make_async_remote_copy(
                    src_ref=ag_ref.at[me],
                    dst_ref=ag_ref.at[me],
                    send_sem=ag_send_sems.at[j],
                    recv_sem=ag_recv_sems.at[me],
                    device_id=(j,),
                    device_id_type=pl.DeviceIdType.MESH,
                ).start()

        for j in range(N_DEV):
            @pl.when(j != me)
            def _():
                pltpu.make_async_remote_copy(
                    src_ref=ag_ref.at[j],
                    dst_ref=ag_ref.at[j],
                    send_sem=ag_send_sems.at[j],
                    recv_sem=ag_recv_sems.at[j],
                    device_id=(j,),
                    device_id_type=pl.DeviceIdType.MESH,
                ).wait_recv()

        out_ref[:, :] = ag_ref[:, :, :].astype(jnp.float32).reshape(SQ, D_MODEL)
        for j in range(N_DEV):
            @pl.when(j == me)
            def _():
                out_ref[j * CHUNK:(j + 1) * CHUNK, :] = red

        for s in range(N_DEV - 1):
            pltpu.make_async_remote_copy(
                src_ref=partial_ref.at[s],
                dst_ref=partial_ref.at[s],
                send_sem=rs_send_sems.at[s],
                recv_sem=rs_recv_sems.at[0],
                device_id=(0,),
                device_id_type=pl.DeviceIdType.MESH,
            ).wait_send()
        for j in range(N_DEV):
            @pl.when(j != me)
            def _():
                pltpu.make_async_remote_copy(
                    src_ref=ag_ref.at[0],
                    dst_ref=ag_ref.at[0],
                    send_sem=ag_send_sems.at[j],
                    recv_sem=ag_recv_sems.at[j],
                    device_id=(j,),
                    device_id_type=pl.DeviceIdType.MESH,
                ).wait_send()

    out = pl.pallas_call(
        body,
        out_shape=jax.ShapeDtypeStruct((SQ, D_MODEL), jnp.float32),
        in_specs=[
            pl.BlockSpec(memory_space=pltpu.VMEM),
            pl.BlockSpec(memory_space=pltpu.MemorySpace.HBM),
            pl.BlockSpec(memory_space=pltpu.MemorySpace.HBM),
            pl.BlockSpec(memory_space=pltpu.MemorySpace.HBM),
            pl.BlockSpec(memory_space=pltpu.MemorySpace.HBM),
        ],
        out_specs=pl.BlockSpec(memory_space=pltpu.VMEM),
        scratch_shapes=[
            pltpu.VMEM((D_MODEL, H_DIM), jnp.float32),
            pltpu.VMEM((H_DIM, D_MODEL), jnp.float32),
            pltpu.VMEM((H_LOCAL, SKV, DH), jnp.float32),
            pltpu.VMEM((H_LOCAL, SKV, DH), jnp.float32),
            pltpu.VMEM((N_DEV - 1, CHUNK, D_MODEL), jnp.bfloat16),
            pltpu.VMEM((N_DEV, CHUNK, D_MODEL), jnp.bfloat16),
            pltpu.VMEM((N_DEV, CHUNK, D_MODEL), jnp.bfloat16),
            pltpu.SemaphoreType.DMA((2 + 2 * H_LOCAL,)),
            pltpu.SemaphoreType.DMA((N_DEV - 1,)),
            pltpu.SemaphoreType.DMA((N_DEV,)),
            pltpu.SemaphoreType.DMA((N_DEV,)),
            pltpu.SemaphoreType.DMA((N_DEV,)),
        ],
        compiler_params=pltpu.CompilerParams(collective_id=0),
    )(x[0], Wq, K_ext, V_ext, Wo)
    return out[None]
